# baseline (speedup 1.0000x reference)
import functools

import jax
import jax.numpy as jnp
from jax import lax
from jax.experimental import pallas as pl
from jax.experimental.pallas import tpu as pltpu
from jax.experimental.pallas import tpu_sc as plsc

E = 320000
HID = 128
N_EDGE_TYPE = 16

_info = plsc.get_sparse_core_info()
_NC, _NS = _info.num_cores, _info.num_subcores
NW = _NC * _NS
R = E // NW
C = 400
NCH = R // C
NGRP = C // 16
GBLK = 5

_mesh = plsc.VectorSubcoreMesh(core_axis_name="c", subcore_axis_name="s")


@functools.partial(
    pl.kernel,
    mesh=_mesh,
    compiler_params=pltpu.CompilerParams(needs_layout_passes=False),
    out_type=jax.ShapeDtypeStruct((E, HID), jnp.float32),
    scratch_types=[
        pltpu.VMEM((N_EDGE_TYPE * HID,), jnp.float32),
        pltpu.VMEM((5 * HID * 16,), jnp.float32),
        pltpu.VMEM((C,), jnp.int32),
        pltpu.VMEM((2 * C,), jnp.float32),
        pltpu.VMEM((3 * C,), jnp.float32),
        pltpu.VMEM((2, C, HID), jnp.float32),
        pltpu.SemaphoreType.DMA,
    ],
)
def _sc_embed(et_hbm, rcT_hbm, rpT_hbm, tbl_hbm, wsp_hbm, out_hbm,
              tbl_v, wsp_v, et_v, rcT_v, rpT_v, out_v, sem):
    wid = lax.axis_index("s") * _NC + lax.axis_index("c")
    base = wid * R
    pltpu.sync_copy(tbl_hbm, tbl_v)
    pltpu.sync_copy(wsp_hbm, wsp_v)
    lanes = lax.broadcasted_iota(jnp.int32, (16,), 0)

    def chunk(i, carry):
        slot = i % 2
        off = base + i * C
        pltpu.sync_copy(et_hbm.at[pl.ds(off, C)], et_v)
        for k in range(2):
            pltpu.sync_copy(rcT_hbm.at[pl.ds(k * E + off, C)],
                            rcT_v.at[pl.ds(k * C, C)])
        for k in range(3):
            pltpu.sync_copy(rpT_hbm.at[pl.ds(k * E + off, C)],
                            rpT_v.at[pl.ds(k * C, C)])

        @pl.when(i >= 2)
        def _():
            pltpu.make_async_copy(out_v.at[0],
                                  out_hbm.at[pl.ds(base, C)], sem).wait()

        slot_splat = jnp.broadcast_to(slot, (16,)).astype(jnp.int32)

        for b in range(NGRP // GBLK):
            et128 = []
            erow = []
            att = []
            for g5 in range(GBLK):
                g = b * GBLK + g5
                et128.append(et_v[pl.ds(g * 16, 16)] * HID)
                erow.append(g * 16 + lanes)
                att.append(
                    [rcT_v[pl.ds(k * C + g * 16, 16)] for k in range(2)]
                    + [rpT_v[pl.ds(k * C + g * 16, 16)] for k in range(3)])

            def dloop(d, carry2):
                wk = [wsp_v[pl.ds(k * HID * 16 + d * 16, 16)]
                      for k in range(5)]
                d_splat = jnp.broadcast_to(d, (16,)).astype(jnp.int32)
                for g5 in range(GBLK):
                    acc = plsc.load_gather(tbl_v, [et128[g5] + d])
                    acc = acc + att[g5][0] * wk[0] + att[g5][1] * wk[1]
                    acc = acc + att[g5][2] * wk[2] + att[g5][3] * wk[3]
                    acc = acc + att[g5][4] * wk[4]
                    plsc.store_scatter(out_v, [slot_splat, erow[g5], d_splat],
                                       acc)
                return carry2

            lax.fori_loop(0, HID, dloop, 0)

        pltpu.async_copy(out_v.at[slot], out_hbm.at[pl.ds(off, C)], sem)
        return carry

    lax.fori_loop(0, NCH, chunk, 0)
    pltpu.make_async_copy(out_v.at[0], out_hbm.at[pl.ds(base, C)], sem).wait()
    pltpu.make_async_copy(out_v.at[1], out_hbm.at[pl.ds(base, C)], sem).wait()


@jax.jit
def kernel(edge_type, att_rc, att_rp, type_table, W_rc, b_rc, W_rp, b_rp):
    tbl2 = (type_table + b_rc + b_rp).reshape(-1)
    w5 = jnp.concatenate([W_rc, W_rp], axis=0)
    wsp = jnp.broadcast_to(w5[:, :, None], (5, HID, 16)).reshape(-1)
    et = edge_type.astype(jnp.int32)
    rcT = att_rc.T.reshape(-1)
    rpT = att_rp.T.reshape(-1)
    return _sc_embed(et, rcT, rpT, tbl2, wsp)


# speedup vs baseline: 6.8756x; 6.8756x over previous
import functools

import jax
import jax.numpy as jnp
from jax import lax
from jax.experimental import pallas as pl

E = 320000
HID = 128
N_EDGE_TYPE = 16
BLOCK = 16000
NB = E // BLOCK


def _body(et_ref, rc_ref, rp_ref, tbl_ref, wrc_ref, brc_ref, wrp_ref,
          brp_ref, out_ref):
    et = et_ref[0, 0, :]
    onehot = (et[:, None] == lax.broadcasted_iota(jnp.int32, (1, N_EDGE_TYPE), 1)
              ).astype(jnp.float32)
    tbl2 = tbl_ref[...] + brc_ref[...] + brp_ref[...]
    acc = jnp.dot(onehot, tbl2, preferred_element_type=jnp.float32)
    acc = acc + jnp.dot(rc_ref[...], wrc_ref[...],
                        preferred_element_type=jnp.float32)
    acc = acc + jnp.dot(rp_ref[...], wrp_ref[...],
                        preferred_element_type=jnp.float32)
    out_ref[...] = acc


@jax.jit
def kernel(edge_type, att_rc, att_rp, type_table, W_rc, b_rc, W_rp, b_rp):
    et3 = edge_type.astype(jnp.int32).reshape(NB, 1, BLOCK)
    brc = b_rc.reshape(1, HID)
    brp = b_rp.reshape(1, HID)
    grid = (NB,)
    return pl.pallas_call(
        _body,
        grid=grid,
        in_specs=[
            pl.BlockSpec((1, 1, BLOCK), lambda i: (i, 0, 0)),
            pl.BlockSpec((BLOCK, 2), lambda i: (i, 0)),
            pl.BlockSpec((BLOCK, 3), lambda i: (i, 0)),
            pl.BlockSpec((N_EDGE_TYPE, HID), lambda i: (0, 0)),
            pl.BlockSpec((2, HID), lambda i: (0, 0)),
            pl.BlockSpec((1, HID), lambda i: (0, 0)),
            pl.BlockSpec((3, HID), lambda i: (0, 0)),
            pl.BlockSpec((1, HID), lambda i: (0, 0)),
        ],
        out_specs=pl.BlockSpec((BLOCK, HID), lambda i: (i, 0)),
        out_shape=jax.ShapeDtypeStruct((E, HID), jnp.float32),
    )(et3, att_rc, att_rp, type_table, W_rc, brc, W_rp, brp)
